# baseline (speedup 1.0000x reference)
import functools

import jax
import jax.numpy as jnp
from jax import lax
from jax.experimental import pallas as pl
from jax.experimental.pallas import tpu as pltpu
from jax.experimental.pallas import tpu_sc as plsc

N = 10000
NP = 10240
E = 320000
D = 128
H = 64
G = 64
NC = 2
NS = 16
NW = NC * NS
TPB = NP // NW
NSEG = 4
SCAP = 3200
SCAPC = SCAP - 16
LISTL = NSEG * SCAP
BLKE = 8000
QBLK = BLKE // NSEG
NBLKP = E // BLKE
GCH = 128
SBLK = 2560
_SC_PARAMS = pltpu.CompilerParams(needs_layout_passes=False,
                                  use_tc_tiling_on_sc=False)

_sc_mesh = plsc.VectorSubcoreMesh(core_axis_name="c", subcore_axis_name="s")


@functools.partial(
    pl.kernel,
    out_type=[
        jax.ShapeDtypeStruct((NW, LISTL), jnp.int32),
        jax.ShapeDtypeStruct((NW, LISTL), jnp.float32),
        jax.ShapeDtypeStruct((NW, LISTL), jnp.int32),
        jax.ShapeDtypeStruct((NW, NSEG * 16), jnp.int32),
        jax.ShapeDtypeStruct((NP,), jnp.float32),
    ],
    mesh=_sc_mesh,
    scratch_types=[
        pltpu.VMEM((LISTL,), jnp.int32),
        pltpu.VMEM((LISTL,), jnp.float32),
        pltpu.VMEM((LISTL,), jnp.int32),
        pltpu.VMEM((TPB,), jnp.float32),
        pltpu.VMEM((2, BLKE), jnp.int32),
        pltpu.VMEM((2, BLKE), jnp.int32),
        pltpu.VMEM((2, BLKE), jnp.float32),
        pltpu.VMEM((NSEG * 16,), jnp.int32),
        pltpu.SemaphoreType.DMA,
        pltpu.SemaphoreType.DMA,
    ],
    compiler_params=_SC_PARAMS,
)
def _sc_prep(src_hbm, dst_hbm, ew_hbm,
             srcs_hbm, ws_hbm, dls_hbm, cnts_hbm, deg_hbm,
             srcl, wl, dll, degl, sb, db, wb, cbuf, sem0, sem1):
    cid = lax.axis_index("c")
    sid = lax.axis_index("s")
    wid = sid * NC + cid
    lo = wid * TPB
    z16 = jnp.zeros((16,), jnp.float32)
    zi16 = jnp.zeros((16,), jnp.int32)
    sems = (sem0, sem1)

    def zfill(i, _):
        srcl[pl.ds(i * 16, 16)] = zi16
        wl[pl.ds(i * 16, 16)] = z16
        dll[pl.ds(i * 16, 16)] = zi16
        return 0
    lax.fori_loop(0, LISTL // 16, zfill, 0)

    def zdeg(i, _):
        degl[pl.ds(i * 16, 16)] = z16
        return 0
    lax.fori_loop(0, TPB // 16, zdeg, 0)

    def issue(t, buf, sem):
        base = t * BLKE
        pltpu.async_copy(src_hbm.at[pl.ds(base, BLKE)], sb.at[buf], sem)
        pltpu.async_copy(dst_hbm.at[pl.ds(base, BLKE)], db.at[buf], sem)
        pltpu.async_copy(ew_hbm.at[pl.ds(base, BLKE)], wb.at[buf], sem)

    def drain(buf, sem):
        pltpu.make_async_copy(src_hbm.at[pl.ds(0, BLKE)], sb.at[buf], sem).wait()
        pltpu.make_async_copy(dst_hbm.at[pl.ds(0, BLKE)], db.at[buf], sem).wait()
        pltpu.make_async_copy(ew_hbm.at[pl.ds(0, BLKE)], wb.at[buf], sem).wait()

    issue(0, 0, sem0)

    def pair(p, cursor):
        curs = cursor
        for q in range(2):
            t = 2 * p + q

            @pl.when(t + 1 < NBLKP)
            def _():
                issue(t + 1, 1 - q, sems[1 - q])

            drain(q, sems[q])

            def vec(j, curs2):
                new = []
                for g4 in range(NSEG):
                    o = (g4 * QBLK // 16 + j) * 16
                    d = db[q, pl.ds(o, 16)]
                    s = sb[q, pl.ds(o, 16)]
                    w = wb[q, pl.ds(o, 16)]
                    m = (d >= lo) & (d < lo + TPB)
                    dlc = jnp.where(m, d - lo, 0)
                    plsc.addupdate_scatter(degl, [dlc], jnp.where(m, w, 0.0))
                    cc = g4 * SCAP + jnp.minimum(curs2[g4], SCAPC)
                    plsc.store_compressed(srcl.at[pl.ds(cc, 16)], s, mask=m)
                    plsc.store_compressed(wl.at[pl.ds(cc, 16)], w, mask=m)
                    plsc.store_compressed(dll.at[pl.ds(cc, 16)], dlc, mask=m)
                    pc = plsc.all_reduce_population_count(m)
                    new.append(curs2[g4] + pc[0])
                return tuple(new)
            curs = lax.fori_loop(0, QBLK // 16, vec, curs)
        return curs
    counts = lax.fori_loop(0, NBLKP // 2, pair,
                           tuple(jnp.int32(0) for _ in range(NSEG)))

    for g4 in range(NSEG):
        cbuf[pl.ds(g4 * 16, 16)] = jnp.full((16,), counts[g4], jnp.int32)
    pltpu.sync_copy(cbuf, cnts_hbm.at[wid])
    pltpu.sync_copy(srcl, srcs_hbm.at[wid])
    pltpu.sync_copy(wl, ws_hbm.at[wid])
    pltpu.sync_copy(dll, dls_hbm.at[wid])
    pltpu.sync_copy(degl, deg_hbm.at[pl.ds(wid * TPB, TPB)])


@functools.partial(
    pl.kernel,
    out_type=jax.ShapeDtypeStruct((NP, H), jnp.float32),
    mesh=_sc_mesh,
    scratch_types=[
        pltpu.VMEM((LISTL,), jnp.int32),
        pltpu.VMEM((LISTL,), jnp.float32),
        pltpu.VMEM((LISTL,), jnp.int32),
        pltpu.VMEM((TPB, H), jnp.float32),
        pltpu.VMEM((2, GCH, H), jnp.float32),
        pltpu.VMEM((NW, NSEG * 16), jnp.int32),
        pltpu.SemaphoreType.DMA,
        pltpu.SemaphoreType.DMA,
    ],
    compiler_params=_SC_PARAMS,
)
def _sc_agg(g_hbm, srcs_hbm, ws_hbm, dls_hbm, cnts_hbm,
            src_hbm, dst_hbm, ew_hbm, out_hbm,
            srcl, wl, dll, acc, rows, cntv, gsem0, gsem1):
    cid = lax.axis_index("c")
    sid = lax.axis_index("s")
    wid = sid * NC + cid
    lo = wid * TPB
    z16 = jnp.zeros((16,), jnp.float32)
    iota = lax.broadcasted_iota(jnp.int32, (16,), 0)
    gsems = (gsem0, gsem1)

    pltpu.sync_copy(cnts_hbm, cntv)
    segc = []
    for g4 in range(NSEG):
        cv = cntv[wid, pl.ds(g4 * 16, 16)]
        segc.append(cv[0])
    cmax = jnp.maximum(jnp.maximum(segc[0], segc[1]),
                       jnp.maximum(segc[2], segc[3]))

    def zacc(r, _):
        for j in range(H // 16):
            acc[r, pl.ds(j * 16, 16)] = z16
        return 0
    lax.fori_loop(0, TPB, zacc, 0)

    def gissue(base, buf):
        pltpu.async_copy(g_hbm.at[srcl.at[pl.ds(base, GCH)]],
                         rows.at[buf], gsems[buf])

    def gdrain(buf):
        pltpu.make_async_copy(g_hbm.at[srcl.at[pl.ds(0, GCH)]],
                              rows.at[buf], gsems[buf]).wait()

    def proc(off, buf):
        def kbbody(kb, _):
            o = off + kb * 16
            dlv = dll[pl.ds(o, 16)]
            wv = wl[pl.ds(o, 16)]
            kvec = iota + kb * 16
            for cj in range(H):
                cjv = jnp.full((16,), cj, jnp.int32)
                vals = plsc.load_gather(rows.at[buf], [kvec, cjv])
                plsc.addupdate_scatter(acc, [dlv, cjv], vals * wv)
            return 0
        lax.fori_loop(0, GCH // 16, kbbody, 0)

    @pl.when(cmax <= SCAPC)
    def _fast():
        for g4 in range(NSEG):
            segbase = g4 * SCAP
            nsub = (segc[g4] + GCH - 1) // GCH

            @pl.when(nsub > 0)
            def _():
                gissue(segbase, 0)

            def pairb(p, _):
                for q in range(2):
                    i = 2 * p + q

                    @pl.when(i < nsub)
                    def _():
                        @pl.when(i + 1 < nsub)
                        def _():
                            gissue(segbase + (i + 1) * GCH, 1 - q)
                        gdrain(q)
                        proc(segbase + i * GCH, q)
                return 0
            lax.fori_loop(0, (nsub + 1) // 2, pairb, 0)

    @pl.when(cmax > SCAPC)
    def _slow():
        def blkbody(t, _):
            base = t * SBLK
            pltpu.sync_copy(src_hbm.at[pl.ds(base, SBLK)],
                            srcl.at[pl.ds(0, SBLK)])
            pltpu.sync_copy(dst_hbm.at[pl.ds(base, SBLK)],
                            dll.at[pl.ds(0, SBLK)])
            pltpu.sync_copy(ew_hbm.at[pl.ds(base, SBLK)],
                            wl.at[pl.ds(0, SBLK)])

            def mv(j, _):
                d = dll[pl.ds(j * 16, 16)]
                s = srcl[pl.ds(j * 16, 16)]
                w = wl[pl.ds(j * 16, 16)]
                m = (d >= lo) & (d < lo + TPB)
                dll[pl.ds(j * 16, 16)] = jnp.where(m, d - lo, 0)
                srcl[pl.ds(j * 16, 16)] = jnp.where(m, s, 0)
                wl[pl.ds(j * 16, 16)] = jnp.where(m, w, 0.0)
                return 0
            lax.fori_loop(0, SBLK // 16, mv, 0)

            def sub(i2, _):
                pltpu.async_copy(g_hbm.at[srcl.at[pl.ds(i2 * GCH, GCH)]],
                                 rows.at[0], gsem0).wait()
                proc(i2 * GCH, 0)
                return 0
            lax.fori_loop(0, SBLK // GCH, sub, 0)
            return 0
        lax.fori_loop(0, E // SBLK, blkbody, 0)

    pltpu.sync_copy(acc, out_hbm.at[pl.ds(wid * TPB, TPB)])


def _tc_a_body(x_ref, w1_ref, deg_ref, g_ref, dinv_ref):
    dinv = lax.rsqrt(deg_ref[...] + 1.0)
    dinv_ref[...] = dinv
    h = jnp.dot(x_ref[...], w1_ref[...], preferred_element_type=jnp.float32)
    g_ref[...] = h * dinv


_tc_a = pl.pallas_call(
    _tc_a_body,
    out_shape=[jax.ShapeDtypeStruct((NP, H), jnp.float32),
               jax.ShapeDtypeStruct((NP, 1), jnp.float32)],
)


def _tc_b_body(acc_ref, g1_ref, dinv_ref, b1_ref, w2_ref, g2_ref):
    dinv = dinv_ref[...]
    o = (acc_ref[...] + g1_ref[...]) * dinv + b1_ref[...]
    o = jnp.maximum(o, 0.0)
    h2 = jnp.dot(o, w2_ref[...], preferred_element_type=jnp.float32)
    g2_ref[...] = h2 * dinv


_tc_b = pl.pallas_call(
    _tc_b_body,
    out_shape=jax.ShapeDtypeStruct((NP, H), jnp.float32),
)


def _tc_c_body(acc_ref, g2_ref, dinv_ref, b2_ref, batch_ref,
               wc1_ref, bc1_ref, wc2_ref, bc2_ref, out_ref):
    o = (acc_ref[...] + g2_ref[...]) * dinv_ref[...] + b2_ref[...]
    o = jnp.maximum(o, 0.0)
    b = batch_ref[...]
    gid = lax.broadcasted_iota(jnp.int32, (G, NP), 0)
    p = (b == gid).astype(jnp.float32)
    s = jnp.dot(p, o, preferred_element_type=jnp.float32)
    cnt = jnp.sum(p, axis=1, keepdims=True)
    mean = s / jnp.maximum(cnt, 1.0)
    z = jnp.dot(mean, wc1_ref[...], preferred_element_type=jnp.float32)
    z = jnp.maximum(z + bc1_ref[...], 0.0)
    out_ref[...] = (jnp.dot(z, wc2_ref[...],
                            preferred_element_type=jnp.float32) + bc2_ref[...])


_tc_c = pl.pallas_call(
    _tc_c_body,
    out_shape=jax.ShapeDtypeStruct((G, 128), jnp.float32),
)


def kernel(x, edge_index, edge_weight, batch, W1, b1, W2, b2, Wc1, bc1, Wc2, bc2):
    src = edge_index[0]
    dst = edge_index[1]
    xp = jnp.pad(x, ((0, NP - N), (0, 0)))
    batch_p = jnp.pad(batch, (0, NP - N), constant_values=-1).reshape(1, NP)

    srcs, ws, dls, cnts, deg = _sc_prep(src, dst, edge_weight)
    g1, dinv = _tc_a(xp, W1, deg.reshape(NP, 1))
    acc1 = _sc_agg(g1, srcs, ws, dls, cnts, src, dst, edge_weight)
    g2 = _tc_b(acc1, g1, dinv, b1.reshape(1, H), W2)
    acc2 = _sc_agg(g2, srcs, ws, dls, cnts, src, dst, edge_weight)

    wc1p = jnp.pad(Wc1, ((0, 0), (0, 128 - H // 2)))
    bc1p = jnp.pad(bc1, (0, 128 - H // 2)).reshape(1, 128)
    wc2p = jnp.pad(Wc2, ((0, 128 - H // 2), (0, 126)))
    bc2p = jnp.pad(bc2, (0, 126)).reshape(1, 128)
    outp = _tc_c(acc2, g2, dinv, b2.reshape(1, H), batch_p,
                 wc1p, bc1p, wc2p, bc2p)
    return outp[:, :2]


# speedup vs baseline: 2.2372x; 2.2372x over previous
import functools

import jax
import jax.numpy as jnp
from jax import lax
from jax.experimental import pallas as pl
from jax.experimental.pallas import tpu as pltpu
from jax.experimental.pallas import tpu_sc as plsc

N = 10000
NP = 10240
E = 320000
D = 128
H = 64
G = 64
NC = 2
NS = 16
NW = NC * NS
TPB = NP // NW
NSEG = 4
SCAP = 3200
SCAPC = SCAP - 16
LISTL = NSEG * SCAP
BLKE = 8000
QBLK = BLKE // NSEG
NBLKP = E // BLKE
GCH = 128
SBLK = 2560
_SC_PARAMS = pltpu.CompilerParams(needs_layout_passes=False,
                                  use_tc_tiling_on_sc=False)

_sc_mesh = plsc.VectorSubcoreMesh(core_axis_name="c", subcore_axis_name="s")


@functools.partial(
    pl.kernel,
    out_type=[
        jax.ShapeDtypeStruct((NW, LISTL), jnp.int32),
        jax.ShapeDtypeStruct((NW, LISTL), jnp.float32),
        jax.ShapeDtypeStruct((NW, LISTL), jnp.int32),
        jax.ShapeDtypeStruct((NW, NSEG * 16), jnp.int32),
        jax.ShapeDtypeStruct((NP,), jnp.float32),
    ],
    mesh=_sc_mesh,
    scratch_types=[
        pltpu.VMEM((LISTL,), jnp.int32),
        pltpu.VMEM((LISTL,), jnp.float32),
        pltpu.VMEM((LISTL,), jnp.int32),
        pltpu.VMEM((TPB,), jnp.float32),
        pltpu.VMEM((2, BLKE), jnp.int32),
        pltpu.VMEM((2, BLKE), jnp.int32),
        pltpu.VMEM((2, BLKE), jnp.float32),
        pltpu.VMEM((NSEG * 16,), jnp.int32),
        pltpu.SemaphoreType.DMA,
        pltpu.SemaphoreType.DMA,
    ],
    compiler_params=_SC_PARAMS,
)
def _sc_prep(src_hbm, dst_hbm, ew_hbm,
             srcs_hbm, ws_hbm, dls_hbm, cnts_hbm, deg_hbm,
             srcl, wl, dll, degl, sb, db, wb, cbuf, sem0, sem1):
    cid = lax.axis_index("c")
    sid = lax.axis_index("s")
    wid = sid * NC + cid
    lo = wid * TPB
    z16 = jnp.zeros((16,), jnp.float32)
    zi16 = jnp.zeros((16,), jnp.int32)
    sems = (sem0, sem1)

    def zfill(i, _):
        srcl[pl.ds(i * 16, 16)] = zi16
        wl[pl.ds(i * 16, 16)] = z16
        dll[pl.ds(i * 16, 16)] = zi16
        return 0
    lax.fori_loop(0, LISTL // 16, zfill, 0)

    def zdeg(i, _):
        degl[pl.ds(i * 16, 16)] = z16
        return 0
    lax.fori_loop(0, TPB // 16, zdeg, 0)

    def issue(t, buf, sem):
        base = t * BLKE
        pltpu.async_copy(src_hbm.at[pl.ds(base, BLKE)], sb.at[buf], sem)
        pltpu.async_copy(dst_hbm.at[pl.ds(base, BLKE)], db.at[buf], sem)
        pltpu.async_copy(ew_hbm.at[pl.ds(base, BLKE)], wb.at[buf], sem)

    def drain(buf, sem):
        pltpu.make_async_copy(src_hbm.at[pl.ds(0, BLKE)], sb.at[buf], sem).wait()
        pltpu.make_async_copy(dst_hbm.at[pl.ds(0, BLKE)], db.at[buf], sem).wait()
        pltpu.make_async_copy(ew_hbm.at[pl.ds(0, BLKE)], wb.at[buf], sem).wait()

    issue(0, 0, sem0)

    def pair(p, cursor):
        curs = cursor
        for q in range(2):
            t = 2 * p + q

            @pl.when(t + 1 < NBLKP)
            def _():
                issue(t + 1, 1 - q, sems[1 - q])

            drain(q, sems[q])

            def vec(j, curs2):
                new = []
                for g4 in range(NSEG):
                    o = (g4 * QBLK // 16 + j) * 16
                    d = db[q, pl.ds(o, 16)]
                    s = sb[q, pl.ds(o, 16)]
                    w = wb[q, pl.ds(o, 16)]
                    m = (d >= lo) & (d < lo + TPB)
                    dlc = jnp.where(m, d - lo, 0)
                    plsc.addupdate_scatter(degl, [dlc], jnp.where(m, w, 0.0))
                    cc = g4 * SCAP + jnp.minimum(curs2[g4], SCAPC)
                    plsc.store_compressed(srcl.at[pl.ds(cc, 16)], s, mask=m)
                    plsc.store_compressed(wl.at[pl.ds(cc, 16)], w, mask=m)
                    plsc.store_compressed(dll.at[pl.ds(cc, 16)], dlc, mask=m)
                    pc = plsc.all_reduce_population_count(m)
                    new.append(curs2[g4] + pc[0])
                return tuple(new)
            curs = lax.fori_loop(0, QBLK // 16, vec, curs)
        return curs
    counts = lax.fori_loop(0, NBLKP // 2, pair,
                           tuple(jnp.int32(0) for _ in range(NSEG)))

    for g4 in range(NSEG):
        cbuf[pl.ds(g4 * 16, 16)] = jnp.full((16,), counts[g4], jnp.int32)
    pltpu.sync_copy(cbuf, cnts_hbm.at[wid])
    pltpu.sync_copy(srcl, srcs_hbm.at[wid])
    pltpu.sync_copy(wl, ws_hbm.at[wid])
    pltpu.sync_copy(dll, dls_hbm.at[wid])
    pltpu.sync_copy(degl, deg_hbm.at[pl.ds(wid * TPB, TPB)])


@functools.partial(
    pl.kernel,
    out_type=jax.ShapeDtypeStruct((NP, H), jnp.float32),
    mesh=_sc_mesh,
    scratch_types=[
        pltpu.VMEM((LISTL,), jnp.int32),
        pltpu.VMEM((LISTL,), jnp.float32),
        pltpu.VMEM((LISTL,), jnp.int32),
        pltpu.VMEM((TPB, H), jnp.float32),
        pltpu.VMEM((2, GCH, H), jnp.float32),
        pltpu.VMEM((NW, NSEG * 16), jnp.int32),
        pltpu.SemaphoreType.DMA,
        pltpu.SemaphoreType.DMA,
    ],
    compiler_params=_SC_PARAMS,
)
def _sc_agg(g_hbm, srcs_hbm, ws_hbm, dls_hbm, cnts_hbm,
            src_hbm, dst_hbm, ew_hbm, out_hbm,
            srcl, wl, dll, acc, rows, cntv, gsem0, gsem1):
    cid = lax.axis_index("c")
    sid = lax.axis_index("s")
    wid = sid * NC + cid
    lo = wid * TPB
    z16 = jnp.zeros((16,), jnp.float32)
    iota = lax.broadcasted_iota(jnp.int32, (16,), 0)
    gsems = (gsem0, gsem1)
    lanesplat = [jnp.full((16, 1), ln, jnp.int32) for ln in range(16)]
    _gdn = lax.GatherDimensionNumbers(offset_dims=(), collapsed_slice_dims=(0,),
                                      start_index_map=(0,))

    def _splat(vec, lane):
        return lax.gather(vec, lanesplat[lane], _gdn, (1,),
                          mode=lax.GatherScatterMode.PROMISE_IN_BOUNDS)

    pltpu.sync_copy(cnts_hbm, cntv)
    segc = []
    for g4 in range(NSEG):
        cv = cntv[wid, pl.ds(g4 * 16, 16)]
        segc.append(cv[0])
    cmax = jnp.maximum(jnp.maximum(segc[0], segc[1]),
                       jnp.maximum(segc[2], segc[3]))

    def zacc(r, _):
        for j in range(H // 16):
            acc[r, pl.ds(j * 16, 16)] = z16
        return 0
    lax.fori_loop(0, TPB, zacc, 0)

    def gissue(base, buf):
        pltpu.async_copy(g_hbm.at[srcl.at[pl.ds(base, GCH)]],
                         rows.at[buf], gsems[buf])

    def gdrain(buf):
        pltpu.make_async_copy(g_hbm.at[srcl.at[pl.ds(0, GCH)]],
                              rows.at[buf], gsems[buf]).wait()

    def proc(off, buf):
        def kbbody(kb, _):
            o = off + kb * 16
            dv = dll[pl.ds(o, 16)]
            wv = wl[pl.ds(o, 16)]
            for lane in range(16):
                d = dv[lane]
                ws = _splat(wv, lane)
                k = kb * 16 + lane
                for j in range(H // 16):
                    plsc.addupdate(acc.at[d, pl.ds(j * 16, 16)],
                                   rows[buf, k, pl.ds(j * 16, 16)] * ws)
            return 0
        lax.fori_loop(0, GCH // 16, kbbody, 0, unroll=2)

    @pl.when(cmax <= SCAPC)
    def _fast():
        for g4 in range(NSEG):
            segbase = g4 * SCAP
            nsub = (segc[g4] + GCH - 1) // GCH

            @pl.when(nsub > 0)
            def _():
                gissue(segbase, 0)

            def pairb(p, _):
                for q in range(2):
                    i = 2 * p + q

                    @pl.when(i < nsub)
                    def _():
                        @pl.when(i + 1 < nsub)
                        def _():
                            gissue(segbase + (i + 1) * GCH, 1 - q)
                        gdrain(q)
                        proc(segbase + i * GCH, q)
                return 0
            lax.fori_loop(0, (nsub + 1) // 2, pairb, 0)

    @pl.when(cmax > SCAPC)
    def _slow():
        def blkbody(t, _):
            base = t * SBLK
            pltpu.sync_copy(src_hbm.at[pl.ds(base, SBLK)],
                            srcl.at[pl.ds(0, SBLK)])
            pltpu.sync_copy(dst_hbm.at[pl.ds(base, SBLK)],
                            dll.at[pl.ds(0, SBLK)])
            pltpu.sync_copy(ew_hbm.at[pl.ds(base, SBLK)],
                            wl.at[pl.ds(0, SBLK)])

            def mv(j, _):
                d = dll[pl.ds(j * 16, 16)]
                s = srcl[pl.ds(j * 16, 16)]
                w = wl[pl.ds(j * 16, 16)]
                m = (d >= lo) & (d < lo + TPB)
                dll[pl.ds(j * 16, 16)] = jnp.where(m, d - lo, 0)
                srcl[pl.ds(j * 16, 16)] = jnp.where(m, s, 0)
                wl[pl.ds(j * 16, 16)] = jnp.where(m, w, 0.0)
                return 0
            lax.fori_loop(0, SBLK // 16, mv, 0)

            def sub(i2, _):
                pltpu.async_copy(g_hbm.at[srcl.at[pl.ds(i2 * GCH, GCH)]],
                                 rows.at[0], gsem0).wait()
                proc(i2 * GCH, 0)
                return 0
            lax.fori_loop(0, SBLK // GCH, sub, 0)
            return 0
        lax.fori_loop(0, E // SBLK, blkbody, 0)

    pltpu.sync_copy(acc, out_hbm.at[pl.ds(wid * TPB, TPB)])


def _tc_a_body(x_ref, w1_ref, deg_ref, g_ref, dinv_ref):
    dinv = lax.rsqrt(deg_ref[...] + 1.0)
    dinv_ref[...] = dinv
    h = jnp.dot(x_ref[...], w1_ref[...], preferred_element_type=jnp.float32)
    g_ref[...] = h * dinv


_tc_a = pl.pallas_call(
    _tc_a_body,
    out_shape=[jax.ShapeDtypeStruct((NP, H), jnp.float32),
               jax.ShapeDtypeStruct((NP, 1), jnp.float32)],
)


def _tc_b_body(acc_ref, g1_ref, dinv_ref, b1_ref, w2_ref, g2_ref):
    dinv = dinv_ref[...]
    o = (acc_ref[...] + g1_ref[...]) * dinv + b1_ref[...]
    o = jnp.maximum(o, 0.0)
    h2 = jnp.dot(o, w2_ref[...], preferred_element_type=jnp.float32)
    g2_ref[...] = h2 * dinv


_tc_b = pl.pallas_call(
    _tc_b_body,
    out_shape=jax.ShapeDtypeStruct((NP, H), jnp.float32),
)


def _tc_c_body(acc_ref, g2_ref, dinv_ref, b2_ref, batch_ref,
               wc1_ref, bc1_ref, wc2_ref, bc2_ref, out_ref):
    o = (acc_ref[...] + g2_ref[...]) * dinv_ref[...] + b2_ref[...]
    o = jnp.maximum(o, 0.0)
    b = batch_ref[...]
    gid = lax.broadcasted_iota(jnp.int32, (G, NP), 0)
    p = (b == gid).astype(jnp.float32)
    s = jnp.dot(p, o, preferred_element_type=jnp.float32)
    cnt = jnp.sum(p, axis=1, keepdims=True)
    mean = s / jnp.maximum(cnt, 1.0)
    z = jnp.dot(mean, wc1_ref[...], preferred_element_type=jnp.float32)
    z = jnp.maximum(z + bc1_ref[...], 0.0)
    out_ref[...] = (jnp.dot(z, wc2_ref[...],
                            preferred_element_type=jnp.float32) + bc2_ref[...])


_tc_c = pl.pallas_call(
    _tc_c_body,
    out_shape=jax.ShapeDtypeStruct((G, 128), jnp.float32),
)


def kernel(x, edge_index, edge_weight, batch, W1, b1, W2, b2, Wc1, bc1, Wc2, bc2):
    src = edge_index[0]
    dst = edge_index[1]
    xp = jnp.pad(x, ((0, NP - N), (0, 0)))
    batch_p = jnp.pad(batch, (0, NP - N), constant_values=-1).reshape(1, NP)

    srcs, ws, dls, cnts, deg = _sc_prep(src, dst, edge_weight)
    g1, dinv = _tc_a(xp, W1, deg.reshape(NP, 1))
    acc1 = _sc_agg(g1, srcs, ws, dls, cnts, src, dst, edge_weight)
    g2 = _tc_b(acc1, g1, dinv, b1.reshape(1, H), W2)
    acc2 = _sc_agg(g2, srcs, ws, dls, cnts, src, dst, edge_weight)

    wc1p = jnp.pad(Wc1, ((0, 0), (0, 128 - H // 2)))
    bc1p = jnp.pad(bc1, (0, 128 - H // 2)).reshape(1, 128)
    wc2p = jnp.pad(Wc2, ((0, 128 - H // 2), (0, 126)))
    bc2p = jnp.pad(bc2, (0, 126)).reshape(1, 128)
    outp = _tc_c(acc2, g2, dinv, b2.reshape(1, H), batch_p,
                 wc1p, bc1p, wc2p, bc2p)
    return outp[:, :2]


# speedup vs baseline: 2.2935x; 1.0252x over previous
import functools

import jax
import jax.numpy as jnp
from jax import lax
from jax.experimental import pallas as pl
from jax.experimental.pallas import tpu as pltpu
from jax.experimental.pallas import tpu_sc as plsc

N = 10000
NP = 10240
E = 320000
D = 128
H = 64
G = 64
NC = 2
NS = 16
NW = NC * NS
TPB = NP // NW
NSEG = 4
SCAP = 3200
SCAPC = SCAP - 16
LISTL = NSEG * SCAP
BLKE = 8000
QBLK = BLKE // NSEG
NBLKP = E // BLKE
GCH = 128
SBLK = 2560
_SC_PARAMS = pltpu.CompilerParams(needs_layout_passes=False,
                                  use_tc_tiling_on_sc=False)

_sc_mesh = plsc.VectorSubcoreMesh(core_axis_name="c", subcore_axis_name="s")


@functools.partial(
    pl.kernel,
    out_type=[
        jax.ShapeDtypeStruct((NW, LISTL), jnp.int32),
        jax.ShapeDtypeStruct((NW, LISTL), jnp.float32),
        jax.ShapeDtypeStruct((NW, LISTL), jnp.int32),
        jax.ShapeDtypeStruct((NW, NSEG * 16), jnp.int32),
        jax.ShapeDtypeStruct((NP,), jnp.float32),
    ],
    mesh=_sc_mesh,
    scratch_types=[
        pltpu.VMEM((LISTL,), jnp.int32),
        pltpu.VMEM((LISTL,), jnp.float32),
        pltpu.VMEM((LISTL,), jnp.int32),
        pltpu.VMEM((TPB,), jnp.float32),
        pltpu.VMEM((2, BLKE), jnp.int32),
        pltpu.VMEM((2, BLKE), jnp.int32),
        pltpu.VMEM((2, BLKE), jnp.float32),
        pltpu.VMEM((NSEG * 16,), jnp.int32),
        pltpu.SemaphoreType.DMA,
        pltpu.SemaphoreType.DMA,
    ],
    compiler_params=_SC_PARAMS,
)
def _sc_prep(src_hbm, dst_hbm, ew_hbm,
             srcs_hbm, ws_hbm, dls_hbm, cnts_hbm, deg_hbm,
             srcl, wl, dll, degl, sb, db, wb, cbuf, sem0, sem1):
    cid = lax.axis_index("c")
    sid = lax.axis_index("s")
    wid = sid * NC + cid
    lo = wid * TPB
    z16 = jnp.zeros((16,), jnp.float32)
    zi16 = jnp.zeros((16,), jnp.int32)
    sems = (sem0, sem1)

    def zfill(i, _):
        srcl[pl.ds(i * 16, 16)] = zi16
        wl[pl.ds(i * 16, 16)] = z16
        dll[pl.ds(i * 16, 16)] = zi16
        return 0
    lax.fori_loop(0, LISTL // 16, zfill, 0)

    def zdeg(i, _):
        degl[pl.ds(i * 16, 16)] = z16
        return 0
    lax.fori_loop(0, TPB // 16, zdeg, 0)

    def issue(t, buf, sem):
        base = t * BLKE
        pltpu.async_copy(src_hbm.at[pl.ds(base, BLKE)], sb.at[buf], sem)
        pltpu.async_copy(dst_hbm.at[pl.ds(base, BLKE)], db.at[buf], sem)
        pltpu.async_copy(ew_hbm.at[pl.ds(base, BLKE)], wb.at[buf], sem)

    def drain(buf, sem):
        pltpu.make_async_copy(src_hbm.at[pl.ds(0, BLKE)], sb.at[buf], sem).wait()
        pltpu.make_async_copy(dst_hbm.at[pl.ds(0, BLKE)], db.at[buf], sem).wait()
        pltpu.make_async_copy(ew_hbm.at[pl.ds(0, BLKE)], wb.at[buf], sem).wait()

    issue(0, 0, sem0)

    def pair(p, cursor):
        curs = cursor
        for q in range(2):
            t = 2 * p + q

            @pl.when(t + 1 < NBLKP)
            def _():
                issue(t + 1, 1 - q, sems[1 - q])

            drain(q, sems[q])

            def vec(j, curs2):
                new = []
                for g4 in range(NSEG):
                    o = (g4 * QBLK // 16 + j) * 16
                    d = db[q, pl.ds(o, 16)]
                    s = sb[q, pl.ds(o, 16)]
                    w = wb[q, pl.ds(o, 16)]
                    m = (d >= lo) & (d < lo + TPB)
                    dlc = jnp.where(m, d - lo, 0)
                    plsc.addupdate_scatter(degl, [dlc], jnp.where(m, w, 0.0))
                    cc = g4 * SCAP + jnp.minimum(curs2[g4], SCAPC)
                    plsc.store_compressed(srcl.at[pl.ds(cc, 16)], s, mask=m)
                    plsc.store_compressed(wl.at[pl.ds(cc, 16)], w, mask=m)
                    plsc.store_compressed(dll.at[pl.ds(cc, 16)], dlc, mask=m)
                    pc = plsc.all_reduce_population_count(m)
                    new.append(curs2[g4] + pc[0])
                return tuple(new)
            curs = lax.fori_loop(0, QBLK // 16, vec, curs)
        return curs
    counts = lax.fori_loop(0, NBLKP // 2, pair,
                           tuple(jnp.int32(0) for _ in range(NSEG)))

    for g4 in range(NSEG):
        cbuf[pl.ds(g4 * 16, 16)] = jnp.full((16,), counts[g4], jnp.int32)
    pltpu.sync_copy(cbuf, cnts_hbm.at[wid])
    pltpu.sync_copy(srcl, srcs_hbm.at[wid])
    pltpu.sync_copy(wl, ws_hbm.at[wid])
    pltpu.sync_copy(dll, dls_hbm.at[wid])
    pltpu.sync_copy(degl, deg_hbm.at[pl.ds(wid * TPB, TPB)])


@functools.partial(
    pl.kernel,
    out_type=jax.ShapeDtypeStruct((NP, H), jnp.float32),
    mesh=_sc_mesh,
    scratch_types=[
        pltpu.VMEM((LISTL,), jnp.int32),
        pltpu.VMEM((LISTL,), jnp.float32),
        pltpu.VMEM((LISTL,), jnp.int32),
        pltpu.VMEM((TPB, H), jnp.float32),
        pltpu.VMEM((4, GCH, H), jnp.float32),
        pltpu.VMEM((NW, NSEG * 16), jnp.int32),
        pltpu.SemaphoreType.DMA,
        pltpu.SemaphoreType.DMA,
        pltpu.SemaphoreType.DMA,
        pltpu.SemaphoreType.DMA,
    ],
    compiler_params=_SC_PARAMS,
)
def _sc_agg(g_hbm, srcs_hbm, ws_hbm, dls_hbm, cnts_hbm,
            src_hbm, dst_hbm, ew_hbm, out_hbm,
            srcl, wl, dll, acc, rows, cntv, gsem0, gsem1, gsem2, gsem3):
    cid = lax.axis_index("c")
    sid = lax.axis_index("s")
    wid = sid * NC + cid
    lo = wid * TPB
    z16 = jnp.zeros((16,), jnp.float32)
    iota = lax.broadcasted_iota(jnp.int32, (16,), 0)
    gsems = (gsem0, gsem1, gsem2, gsem3)
    lanesplat = [jnp.full((16, 1), ln, jnp.int32) for ln in range(16)]
    _gdn = lax.GatherDimensionNumbers(offset_dims=(), collapsed_slice_dims=(0,),
                                      start_index_map=(0,))

    def _splat(vec, lane):
        return lax.gather(vec, lanesplat[lane], _gdn, (1,),
                          mode=lax.GatherScatterMode.PROMISE_IN_BOUNDS)

    pltpu.sync_copy(cnts_hbm, cntv)
    segc = []
    for g4 in range(NSEG):
        cv = cntv[wid, pl.ds(g4 * 16, 16)]
        segc.append(cv[0])
    cmax = jnp.maximum(jnp.maximum(segc[0], segc[1]),
                       jnp.maximum(segc[2], segc[3]))

    def zacc(r, _):
        for j in range(H // 16):
            acc[r, pl.ds(j * 16, 16)] = z16
        return 0
    lax.fori_loop(0, TPB, zacc, 0)

    def gissue(base, buf):
        pltpu.async_copy(g_hbm.at[srcl.at[pl.ds(base, GCH)]],
                         rows.at[buf], gsems[buf])

    def gdrain(buf):
        pltpu.make_async_copy(g_hbm.at[srcl.at[pl.ds(0, GCH)]],
                              rows.at[buf], gsems[buf]).wait()

    def proc(off, buf):
        def kbbody(kb, _):
            o = off + kb * 16
            dv = dll[pl.ds(o, 16)]
            wv = wl[pl.ds(o, 16)]
            for lane in range(16):
                d = dv[lane]
                ws = _splat(wv, lane)
                k = kb * 16 + lane
                for j in range(H // 16):
                    plsc.addupdate(acc.at[d, pl.ds(j * 16, 16)],
                                   rows[buf, k, pl.ds(j * 16, 16)] * ws)
            return 0
        lax.fori_loop(0, GCH // 16, kbbody, 0)

    @pl.when(cmax <= SCAPC)
    def _fast():
        for g4 in range(NSEG):
            segbase = g4 * SCAP
            nsub = (segc[g4] + GCH - 1) // GCH

            for b in range(3):
                @pl.when(b < nsub)
                def _(b=b):
                    gissue(segbase + b * GCH, b)

            def quadb(p, _):
                for q in range(4):
                    i = 4 * p + q

                    @pl.when(i < nsub)
                    def _():
                        @pl.when(i + 3 < nsub)
                        def _():
                            gissue(segbase + (i + 3) * GCH, (q + 3) % 4)
                        gdrain(q)
                        proc(segbase + i * GCH, q)
                return 0
            lax.fori_loop(0, (nsub + 3) // 4, quadb, 0)

    @pl.when(cmax > SCAPC)
    def _slow():
        def blkbody(t, _):
            base = t * SBLK
            pltpu.sync_copy(src_hbm.at[pl.ds(base, SBLK)],
                            srcl.at[pl.ds(0, SBLK)])
            pltpu.sync_copy(dst_hbm.at[pl.ds(base, SBLK)],
                            dll.at[pl.ds(0, SBLK)])
            pltpu.sync_copy(ew_hbm.at[pl.ds(base, SBLK)],
                            wl.at[pl.ds(0, SBLK)])

            def mv(j, _):
                d = dll[pl.ds(j * 16, 16)]
                s = srcl[pl.ds(j * 16, 16)]
                w = wl[pl.ds(j * 16, 16)]
                m = (d >= lo) & (d < lo + TPB)
                dll[pl.ds(j * 16, 16)] = jnp.where(m, d - lo, 0)
                srcl[pl.ds(j * 16, 16)] = jnp.where(m, s, 0)
                wl[pl.ds(j * 16, 16)] = jnp.where(m, w, 0.0)
                return 0
            lax.fori_loop(0, SBLK // 16, mv, 0)

            def sub(i2, _):
                pltpu.async_copy(g_hbm.at[srcl.at[pl.ds(i2 * GCH, GCH)]],
                                 rows.at[0], gsem0).wait()
                proc(i2 * GCH, 0)
                return 0
            lax.fori_loop(0, SBLK // GCH, sub, 0)
            return 0
        lax.fori_loop(0, E // SBLK, blkbody, 0)

    pltpu.sync_copy(acc, out_hbm.at[pl.ds(wid * TPB, TPB)])


def _tc_a_body(x_ref, w1_ref, deg_ref, g_ref, dinv_ref):
    dinv = lax.rsqrt(deg_ref[...] + 1.0)
    dinv_ref[...] = dinv
    h = jnp.dot(x_ref[...], w1_ref[...], preferred_element_type=jnp.float32)
    g_ref[...] = h * dinv


_tc_a = pl.pallas_call(
    _tc_a_body,
    out_shape=[jax.ShapeDtypeStruct((NP, H), jnp.float32),
               jax.ShapeDtypeStruct((NP, 1), jnp.float32)],
)


def _tc_b_body(acc_ref, g1_ref, dinv_ref, b1_ref, w2_ref, g2_ref):
    dinv = dinv_ref[...]
    o = (acc_ref[...] + g1_ref[...]) * dinv + b1_ref[...]
    o = jnp.maximum(o, 0.0)
    h2 = jnp.dot(o, w2_ref[...], preferred_element_type=jnp.float32)
    g2_ref[...] = h2 * dinv


_tc_b = pl.pallas_call(
    _tc_b_body,
    out_shape=jax.ShapeDtypeStruct((NP, H), jnp.float32),
)


def _tc_c_body(acc_ref, g2_ref, dinv_ref, b2_ref, batch_ref,
               wc1_ref, bc1_ref, wc2_ref, bc2_ref, out_ref):
    o = (acc_ref[...] + g2_ref[...]) * dinv_ref[...] + b2_ref[...]
    o = jnp.maximum(o, 0.0)
    b = batch_ref[...]
    gid = lax.broadcasted_iota(jnp.int32, (G, NP), 0)
    p = (b == gid).astype(jnp.float32)
    s = jnp.dot(p, o, preferred_element_type=jnp.float32)
    cnt = jnp.sum(p, axis=1, keepdims=True)
    mean = s / jnp.maximum(cnt, 1.0)
    z = jnp.dot(mean, wc1_ref[...], preferred_element_type=jnp.float32)
    z = jnp.maximum(z + bc1_ref[...], 0.0)
    out_ref[...] = (jnp.dot(z, wc2_ref[...],
                            preferred_element_type=jnp.float32) + bc2_ref[...])


_tc_c = pl.pallas_call(
    _tc_c_body,
    out_shape=jax.ShapeDtypeStruct((G, 128), jnp.float32),
)


def kernel(x, edge_index, edge_weight, batch, W1, b1, W2, b2, Wc1, bc1, Wc2, bc2):
    src = edge_index[0]
    dst = edge_index[1]
    xp = jnp.pad(x, ((0, NP - N), (0, 0)))
    batch_p = jnp.pad(batch, (0, NP - N), constant_values=-1).reshape(1, NP)

    srcs, ws, dls, cnts, deg = _sc_prep(src, dst, edge_weight)
    g1, dinv = _tc_a(xp, W1, deg.reshape(NP, 1))
    acc1 = _sc_agg(g1, srcs, ws, dls, cnts, src, dst, edge_weight)
    g2 = _tc_b(acc1, g1, dinv, b1.reshape(1, H), W2)
    acc2 = _sc_agg(g2, srcs, ws, dls, cnts, src, dst, edge_weight)

    wc1p = jnp.pad(Wc1, ((0, 0), (0, 128 - H // 2)))
    bc1p = jnp.pad(bc1, (0, 128 - H // 2)).reshape(1, 128)
    wc2p = jnp.pad(Wc2, ((0, 128 - H // 2), (0, 126)))
    bc2p = jnp.pad(bc2, (0, 126)).reshape(1, 128)
    outp = _tc_c(acc2, g2, dinv, b2.reshape(1, H), batch_p,
                 wc1p, bc1p, wc2p, bc2p)
    return outp[:, :2]


# speedup vs baseline: 2.6735x; 1.1657x over previous
import functools

import jax
import jax.numpy as jnp
from jax import lax
from jax.experimental import pallas as pl
from jax.experimental.pallas import tpu as pltpu
from jax.experimental.pallas import tpu_sc as plsc

N = 10000
NP = 10240
E = 320000
D = 128
H = 64
G = 64
NC = 2
NS = 16
NW = NC * NS
TPB = NP // NW
NSEG = 4
SCAP = 3200
SCAPC = SCAP - 16
LISTL = NSEG * SCAP
BLKE = 8000
QBLK = BLKE // NSEG
NBLKP = E // BLKE
GCH = 128
SBLK = 2560
_SC_PARAMS = pltpu.CompilerParams(needs_layout_passes=False,
                                  use_tc_tiling_on_sc=False)

_sc_mesh = plsc.VectorSubcoreMesh(core_axis_name="c", subcore_axis_name="s")


@functools.partial(
    pl.kernel,
    out_type=[
        jax.ShapeDtypeStruct((NW, LISTL), jnp.int32),
        jax.ShapeDtypeStruct((NW, LISTL), jnp.float32),
        jax.ShapeDtypeStruct((NW, LISTL), jnp.int32),
        jax.ShapeDtypeStruct((NW, NSEG * 16), jnp.int32),
        jax.ShapeDtypeStruct((NP,), jnp.float32),
    ],
    mesh=_sc_mesh,
    scratch_types=[
        pltpu.VMEM((LISTL,), jnp.int32),
        pltpu.VMEM((LISTL,), jnp.float32),
        pltpu.VMEM((LISTL,), jnp.int32),
        pltpu.VMEM((TPB,), jnp.float32),
        pltpu.VMEM((2, BLKE), jnp.int32),
        pltpu.VMEM((2, BLKE), jnp.int32),
        pltpu.VMEM((2, BLKE), jnp.float32),
        pltpu.VMEM((NSEG * 16,), jnp.int32),
        pltpu.SemaphoreType.DMA,
        pltpu.SemaphoreType.DMA,
    ],
    compiler_params=_SC_PARAMS,
)
def _sc_prep(src_hbm, dst_hbm, ew_hbm,
             srcs_hbm, ws_hbm, dls_hbm, cnts_hbm, deg_hbm,
             srcl, wl, dll, degl, sb, db, wb, cbuf, sem0, sem1):
    cid = lax.axis_index("c")
    sid = lax.axis_index("s")
    wid = sid * NC + cid
    lo = wid * TPB
    z16 = jnp.zeros((16,), jnp.float32)
    zi16 = jnp.zeros((16,), jnp.int32)
    sems = (sem0, sem1)

    def zfill(i, _):
        srcl[pl.ds(i * 16, 16)] = zi16
        wl[pl.ds(i * 16, 16)] = z16
        dll[pl.ds(i * 16, 16)] = zi16
        return 0
    lax.fori_loop(0, LISTL // 16, zfill, 0)

    def zdeg(i, _):
        degl[pl.ds(i * 16, 16)] = z16
        return 0
    lax.fori_loop(0, TPB // 16, zdeg, 0)

    def issue(t, buf, sem):
        base = t * BLKE
        pltpu.async_copy(src_hbm.at[pl.ds(base, BLKE)], sb.at[buf], sem)
        pltpu.async_copy(dst_hbm.at[pl.ds(base, BLKE)], db.at[buf], sem)
        pltpu.async_copy(ew_hbm.at[pl.ds(base, BLKE)], wb.at[buf], sem)

    def drain(buf, sem):
        pltpu.make_async_copy(src_hbm.at[pl.ds(0, BLKE)], sb.at[buf], sem).wait()
        pltpu.make_async_copy(dst_hbm.at[pl.ds(0, BLKE)], db.at[buf], sem).wait()
        pltpu.make_async_copy(ew_hbm.at[pl.ds(0, BLKE)], wb.at[buf], sem).wait()

    issue(0, 0, sem0)

    def pair(p, cursor):
        curs = cursor
        for q in range(2):
            t = 2 * p + q

            @pl.when(t + 1 < NBLKP)
            def _():
                issue(t + 1, 1 - q, sems[1 - q])

            drain(q, sems[q])

            def vec(j, curs2):
                new = []
                for g4 in range(NSEG):
                    o = (g4 * QBLK // 16 + j) * 16
                    d = db[q, pl.ds(o, 16)]
                    s = sb[q, pl.ds(o, 16)]
                    w = wb[q, pl.ds(o, 16)]
                    m = (d >= lo) & (d < lo + TPB)
                    dlc = jnp.where(m, d - lo, 0)
                    plsc.addupdate_scatter(degl, [dlc], jnp.where(m, w, 0.0))
                    cc = g4 * SCAP + jnp.minimum(curs2[g4], SCAPC)
                    plsc.store_compressed(srcl.at[pl.ds(cc, 16)], s, mask=m)
                    plsc.store_compressed(wl.at[pl.ds(cc, 16)], w, mask=m)
                    plsc.store_compressed(dll.at[pl.ds(cc, 16)], dlc, mask=m)
                    pc = plsc.all_reduce_population_count(m)
                    new.append(curs2[g4] + pc[0])
                return tuple(new)
            curs = lax.fori_loop(0, QBLK // 16, vec, curs)
        return curs
    counts = lax.fori_loop(0, NBLKP // 2, pair,
                           tuple(jnp.int32(0) for _ in range(NSEG)))

    for g4 in range(NSEG):
        cbuf[pl.ds(g4 * 16, 16)] = jnp.full((16,), counts[g4], jnp.int32)
    pltpu.sync_copy(cbuf, cnts_hbm.at[wid])
    pltpu.sync_copy(srcl, srcs_hbm.at[wid])
    pltpu.sync_copy(wl, ws_hbm.at[wid])
    pltpu.sync_copy(dll, dls_hbm.at[wid])
    pltpu.sync_copy(degl, deg_hbm.at[pl.ds(wid * TPB, TPB)])


@functools.partial(
    pl.kernel,
    out_type=jax.ShapeDtypeStruct((NP, H), jnp.float32),
    mesh=_sc_mesh,
    scratch_types=[
        pltpu.VMEM((LISTL,), jnp.int32),
        pltpu.VMEM((LISTL,), jnp.float32),
        pltpu.VMEM((LISTL,), jnp.int32),
        pltpu.VMEM((TPB, H), jnp.float32),
        pltpu.VMEM((4, GCH, H), jnp.bfloat16),
        pltpu.VMEM((NW, NSEG * 16), jnp.int32),
        pltpu.SemaphoreType.DMA,
        pltpu.SemaphoreType.DMA,
        pltpu.SemaphoreType.DMA,
        pltpu.SemaphoreType.DMA,
    ],
    compiler_params=_SC_PARAMS,
)
def _sc_agg(g_hbm, srcs_hbm, ws_hbm, dls_hbm, cnts_hbm,
            src_hbm, dst_hbm, ew_hbm, out_hbm,
            srcl, wl, dll, acc, rows, cntv, gsem0, gsem1, gsem2, gsem3):
    cid = lax.axis_index("c")
    sid = lax.axis_index("s")
    wid = sid * NC + cid
    lo = wid * TPB
    z16 = jnp.zeros((16,), jnp.float32)
    iota = lax.broadcasted_iota(jnp.int32, (16,), 0)
    gsems = (gsem0, gsem1, gsem2, gsem3)
    lanesplat = [jnp.full((16, 1), ln, jnp.int32) for ln in range(16)]
    _gdn = lax.GatherDimensionNumbers(offset_dims=(), collapsed_slice_dims=(0,),
                                      start_index_map=(0,))

    def _splat(vec, lane):
        return lax.gather(vec, lanesplat[lane], _gdn, (1,),
                          mode=lax.GatherScatterMode.PROMISE_IN_BOUNDS)

    pltpu.sync_copy(cnts_hbm, cntv)
    segc = []
    for g4 in range(NSEG):
        cv = cntv[wid, pl.ds(g4 * 16, 16)]
        segc.append(cv[0])
    cmax = jnp.maximum(jnp.maximum(segc[0], segc[1]),
                       jnp.maximum(segc[2], segc[3]))

    def zacc(r, _):
        for j in range(H // 16):
            acc[r, pl.ds(j * 16, 16)] = z16
        return 0
    lax.fori_loop(0, TPB, zacc, 0)

    def gissue(base, buf):
        pltpu.async_copy(g_hbm.at[srcl.at[pl.ds(base, GCH)]],
                         rows.at[buf], gsems[buf])

    def gdrain(buf):
        pltpu.make_async_copy(g_hbm.at[srcl.at[pl.ds(0, GCH)]],
                              rows.at[buf], gsems[buf]).wait()

    def proc(off, buf):
        mhi = jnp.full((16,), -65536, jnp.int32)

        def kbbody(kb, _):
            o = off + kb * 16
            dv = dll[pl.ds(o, 16)]
            wv = wl[pl.ds(o, 16)]
            for lane in range(16):
                d = dv[lane]
                ws = _splat(wv, lane)
                k = kb * 16 + lane
                for half in range(H // 32):
                    v32 = rows[buf, k, pl.ds(half * 32, 32)]
                    w16 = plsc.bitcast(v32, jnp.int32)
                    lo = plsc.bitcast(w16 << 16, jnp.float32)
                    hi = plsc.bitcast(w16 & mhi, jnp.float32)
                    plsc.addupdate(acc.at[d, pl.ds(half * 32, 16)], lo * ws)
                    plsc.addupdate(acc.at[d, pl.ds(half * 32 + 16, 16)],
                                   hi * ws)
            return 0
        lax.fori_loop(0, GCH // 16, kbbody, 0)

    @pl.when(cmax <= SCAPC)
    def _fast():
        for g4 in range(NSEG):
            segbase = g4 * SCAP
            nsub = (segc[g4] + GCH - 1) // GCH

            for b in range(3):
                @pl.when(b < nsub)
                def _(b=b):
                    gissue(segbase + b * GCH, b)

            def quadb(p, _):
                for q in range(4):
                    i = 4 * p + q

                    @pl.when(i < nsub)
                    def _():
                        @pl.when(i + 3 < nsub)
                        def _():
                            gissue(segbase + (i + 3) * GCH, (q + 3) % 4)
                        gdrain(q)
                        proc(segbase + i * GCH, q)
                return 0
            lax.fori_loop(0, (nsub + 3) // 4, quadb, 0)

    @pl.when(cmax > SCAPC)
    def _slow():
        def blkbody(t, _):
            base = t * SBLK
            pltpu.sync_copy(src_hbm.at[pl.ds(base, SBLK)],
                            srcl.at[pl.ds(0, SBLK)])
            pltpu.sync_copy(dst_hbm.at[pl.ds(base, SBLK)],
                            dll.at[pl.ds(0, SBLK)])
            pltpu.sync_copy(ew_hbm.at[pl.ds(base, SBLK)],
                            wl.at[pl.ds(0, SBLK)])

            def mv(j, _):
                d = dll[pl.ds(j * 16, 16)]
                s = srcl[pl.ds(j * 16, 16)]
                w = wl[pl.ds(j * 16, 16)]
                m = (d >= lo) & (d < lo + TPB)
                dll[pl.ds(j * 16, 16)] = jnp.where(m, d - lo, 0)
                srcl[pl.ds(j * 16, 16)] = jnp.where(m, s, 0)
                wl[pl.ds(j * 16, 16)] = jnp.where(m, w, 0.0)
                return 0
            lax.fori_loop(0, SBLK // 16, mv, 0)

            def sub(i2, _):
                pltpu.async_copy(g_hbm.at[srcl.at[pl.ds(i2 * GCH, GCH)]],
                                 rows.at[0], gsem0).wait()
                proc(i2 * GCH, 0)
                return 0
            lax.fori_loop(0, SBLK // GCH, sub, 0)
            return 0
        lax.fori_loop(0, E // SBLK, blkbody, 0)

    pltpu.sync_copy(acc, out_hbm.at[pl.ds(wid * TPB, TPB)])


def _tc_a_body(x_ref, w1_ref, w1t_ref, deg_ref, g_ref, gt_ref, dinv_ref):
    dinv = lax.rsqrt(deg_ref[...] + 1.0)
    dinv_ref[...] = dinv
    h = jnp.dot(x_ref[...], w1_ref[...], preferred_element_type=jnp.float32)
    g_ref[...] = (h * dinv).astype(jnp.bfloat16)
    ht = jnp.dot(x_ref[...], w1t_ref[...], preferred_element_type=jnp.float32)
    gt_ref[...] = ht * dinv


_tc_a = pl.pallas_call(
    _tc_a_body,
    out_shape=[jax.ShapeDtypeStruct((NP, H), jnp.bfloat16),
               jax.ShapeDtypeStruct((NP, H), jnp.float32),
               jax.ShapeDtypeStruct((NP, 1), jnp.float32)],
)


def _tc_b_body(acc_ref, g1t_ref, dinv_ref, b1_ref, w2t_ref, w2tt_ref,
               g2_ref, g2t_ref):
    dinv = dinv_ref[...]
    o = (acc_ref[...] + g1t_ref[...]) * dinv + b1_ref[...]
    o = jnp.maximum(o, 0.0)
    h2 = jnp.dot(o, w2t_ref[...], preferred_element_type=jnp.float32)
    g2_ref[...] = (h2 * dinv).astype(jnp.bfloat16)
    h2t = jnp.dot(o, w2tt_ref[...], preferred_element_type=jnp.float32)
    g2t_ref[...] = h2t * dinv


_tc_b = pl.pallas_call(
    _tc_b_body,
    out_shape=[jax.ShapeDtypeStruct((NP, H), jnp.bfloat16),
               jax.ShapeDtypeStruct((NP, H), jnp.float32)],
)


def _tc_c_body(acc_ref, g2_ref, dinv_ref, b2_ref, batch_ref,
               wc1_ref, bc1_ref, wc2_ref, bc2_ref, out_ref):
    o = (acc_ref[...] + g2_ref[...]) * dinv_ref[...] + b2_ref[...]
    o = jnp.maximum(o, 0.0)
    b = batch_ref[...]
    gid = lax.broadcasted_iota(jnp.int32, (G, NP), 0)
    p = (b == gid).astype(jnp.float32)
    s = jnp.dot(p, o, preferred_element_type=jnp.float32)
    cnt = jnp.sum(p, axis=1, keepdims=True)
    mean = s / jnp.maximum(cnt, 1.0)
    z = jnp.dot(mean, wc1_ref[...], preferred_element_type=jnp.float32)
    z = jnp.maximum(z + bc1_ref[...], 0.0)
    out_ref[...] = (jnp.dot(z, wc2_ref[...],
                            preferred_element_type=jnp.float32) + bc2_ref[...])


_tc_c = pl.pallas_call(
    _tc_c_body,
    out_shape=jax.ShapeDtypeStruct((G, 128), jnp.float32),
)


def kernel(x, edge_index, edge_weight, batch, W1, b1, W2, b2, Wc1, bc1, Wc2, bc2):
    src = edge_index[0]
    dst = edge_index[1]
    xp = jnp.pad(x, ((0, NP - N), (0, 0)))
    batch_p = jnp.pad(batch, (0, NP - N), constant_values=-1).reshape(1, NP)

    tau = jnp.array([*range(0, 32, 2), *range(1, 32, 2),
                     *range(32, 64, 2), *range(33, 64, 2)], dtype=jnp.int32)

    srcs, ws, dls, cnts, deg = _sc_prep(src, dst, edge_weight)
    g1, g1t, dinv = _tc_a(xp, W1, W1[:, tau], deg.reshape(NP, 1))
    acc1 = _sc_agg(g1, srcs, ws, dls, cnts, src, dst, edge_weight)
    g2, g2t = _tc_b(acc1, g1t, dinv, b1[tau].reshape(1, H),
                    W2[tau, :], W2[tau][:, tau])
    acc2 = _sc_agg(g2, srcs, ws, dls, cnts, src, dst, edge_weight)

    wc1p = jnp.pad(Wc1[tau, :], ((0, 0), (0, 128 - H // 2)))
    bc1p = jnp.pad(bc1, (0, 128 - H // 2)).reshape(1, 128)
    wc2p = jnp.pad(Wc2, ((0, 128 - H // 2), (0, 126)))
    bc2p = jnp.pad(bc2, (0, 126)).reshape(1, 128)
    outp = _tc_c(acc2, g2t, dinv, b2[tau].reshape(1, H), batch_p,
                 wc1p, bc1p, wc2p, bc2p)
    return outp[:, :2]
